# Initial kernel scaffold; baseline (speedup 1.0000x reference)
import jax
import jax.numpy as jnp
from jax.experimental import pallas as pl


def kernel(x, weight, bias):
    raise NotImplementedError("write your pallas kernel here")


# speedup vs baseline: 5.7629x; 5.7629x over previous
import functools

import numpy as np

import jax
import jax.numpy as jnp
from jax.experimental import pallas as pl
from jax.experimental.pallas import tpu as pltpu

B, CIN, COUT, T, KW = 16, 64, 64, 4096, 64
FQ = 1024
TOUT = 2 * (FQ - 1)
TPAD = 2048

OB = 8
FB = 512


@functools.lru_cache(maxsize=1)
def _bases():
    t = np.arange(T, dtype=np.float64)[:, None]
    f = np.arange(FQ, dtype=np.float64)[None, :]
    ang = 2.0 * np.pi * t * f / T
    fwd = np.concatenate([np.cos(ang), -np.sin(ang)], axis=1)

    k = np.arange(KW, dtype=np.float64)[:, None]
    angw = 2.0 * np.pi * (T - KW + k) * f / T
    fwdw = np.concatenate([np.cos(angw), -np.sin(angw)], axis=1)

    tt = np.arange(TOUT, dtype=np.float64)[None, :]
    fi = np.arange(FQ, dtype=np.float64)[:, None]
    angi = 2.0 * np.pi * fi * tt / TOUT
    alpha = np.full((FQ, 1), 2.0)
    alpha[0, 0] = 1.0
    alpha[FQ - 1, 0] = 1.0
    ar = alpha * np.cos(angi) / TOUT
    ai = -2.0 * np.sin(angi) / TOUT
    ai[0, :] = 0.0
    ai[FQ - 1, :] = 0.0
    inv = np.concatenate([ar, ai], axis=0)
    inv = np.pad(inv, ((0, 0), (0, TPAD - TOUT)))
    return (fwd.astype(np.float32), fwdw.astype(np.float32),
            inv.astype(np.float32))


def _matmul_kernel(x_ref, w_ref, o_ref):
    o_ref[...] = jnp.dot(x_ref[...], w_ref[...],
                         preferred_element_type=jnp.float32)


def _mm(x, w, bm, bn, out_dtype=jnp.float32):
    m, k = x.shape
    _, n = w.shape
    return pl.pallas_call(
        _matmul_kernel,
        grid=(m // bm, n // bn),
        in_specs=[
            pl.BlockSpec((bm, k), lambda i, j: (i, 0)),
            pl.BlockSpec((k, bn), lambda i, j: (0, j)),
        ],
        out_specs=pl.BlockSpec((bm, bn), lambda i, j: (i, j)),
        out_shape=jax.ShapeDtypeStruct((m, n), out_dtype),
        compiler_params=pltpu.CompilerParams(
            dimension_semantics=("parallel", "arbitrary"),
        ),
    )(x, w)


def _hadamard_kernel(xr_ref, xi_ref, wr_ref, wi_ref, cr_ref, ci_ref):
    for oj in range(OB):
        def body(c, accs):
            acc_r, acc_i = accs
            xr = xr_ref[c]
            xi = xi_ref[c]
            wr = wr_ref[oj * CIN + c]
            wi = wi_ref[oj * CIN + c]
            acc_r = acc_r + xr * wr - xi * wi
            acc_i = acc_i + xr * wi + xi * wr
            return acc_r, acc_i

        zero = jnp.zeros((B, FB), jnp.float32)
        acc_r, acc_i = jax.lax.fori_loop(0, CIN, body, (zero, zero))
        cr_ref[oj] = acc_r.astype(cr_ref.dtype)
        ci_ref[oj] = acc_i.astype(ci_ref.dtype)


def _hadamard(xhat, what):
    xv = xhat.reshape(CIN, B, 2 * FQ)
    wv = what.reshape(COUT * CIN, 1, 2 * FQ)
    nf = FQ // FB
    grid = (COUT // OB, nf)
    xspec = lambda off: pl.BlockSpec(
        (CIN, B, FB), lambda oi, fi: (0, 0, fi + off))
    wspec = lambda off: pl.BlockSpec(
        (OB * CIN, 1, FB), lambda oi, fi: (oi, 0, fi + off))
    ospec = pl.BlockSpec((OB, B, FB), lambda oi, fi: (oi, 0, fi))
    oshape = jax.ShapeDtypeStruct((COUT, B, FQ), jnp.bfloat16)
    return pl.pallas_call(
        _hadamard_kernel,
        grid=grid,
        in_specs=[xspec(0), xspec(nf), wspec(0), wspec(nf)],
        out_specs=[ospec, ospec],
        out_shape=[oshape, oshape],
        compiler_params=pltpu.CompilerParams(
            dimension_semantics=("parallel", "arbitrary"),
        ),
    )(xv, xv, wv, wv)


def _inverse_kernel(cr_ref, ci_ref, ar_ref, ai_ref, b_ref, o_ref):
    acc = jnp.dot(cr_ref[...], ar_ref[...],
                  preferred_element_type=jnp.float32)
    acc += jnp.dot(ci_ref[...], ai_ref[...],
                   preferred_element_type=jnp.float32)
    o_ref[...] = acc + pltpu.repeat(b_ref[...], o_ref.shape[1] // 128, axis=1)


def _inverse(cr, ci, inv_r, inv_i, bias_plane):
    m = COUT * B
    bm, bn = 512, 1024
    return pl.pallas_call(
        _inverse_kernel,
        grid=(m // bm, TPAD // bn),
        in_specs=[
            pl.BlockSpec((bm, FQ), lambda i, j: (i, 0)),
            pl.BlockSpec((bm, FQ), lambda i, j: (i, 0)),
            pl.BlockSpec((FQ, bn), lambda i, j: (0, j)),
            pl.BlockSpec((FQ, bn), lambda i, j: (0, j)),
            pl.BlockSpec((bm, 128), lambda i, j: (i, 0)),
        ],
        out_specs=pl.BlockSpec((bm, bn), lambda i, j: (i, j)),
        out_shape=jax.ShapeDtypeStruct((m, TPAD), jnp.float32),
        compiler_params=pltpu.CompilerParams(
            dimension_semantics=("parallel", "arbitrary"),
        ),
    )(cr, ci, inv_r, inv_i, bias_plane)


def kernel(x, weight, bias):
    fwd, fwdw, inv = _bases()
    fwd = jnp.asarray(fwd, jnp.bfloat16)
    fwdw = jnp.asarray(fwdw, jnp.bfloat16)
    inv_r = jnp.asarray(inv[:FQ], jnp.bfloat16)
    inv_i = jnp.asarray(inv[FQ:], jnp.bfloat16)

    xt = x.transpose(1, 0, 2).reshape(CIN * B, T).astype(jnp.bfloat16)
    xhat = _mm(xt, fwd, bm=512, bn=1024)

    w2 = weight.reshape(COUT * CIN, KW).astype(jnp.bfloat16)
    what = _mm(w2, fwdw, bm=1024, bn=2048)

    cr, ci = _hadamard(xhat, what)

    bias_plane = jnp.broadcast_to(
        jnp.repeat(bias, B)[:, None], (COUT * B, 128)).astype(jnp.float32)
    out = _inverse(cr.reshape(COUT * B, FQ), ci.reshape(COUT * B, FQ),
                   inv_r, inv_i, bias_plane)

    return out[:, :TOUT].reshape(COUT, B, TOUT).transpose(1, 0, 2)


# speedup vs baseline: 6.7177x; 1.1657x over previous
import functools

import numpy as np

import jax
import jax.numpy as jnp
from jax.experimental import pallas as pl
from jax.experimental.pallas import tpu as pltpu

B, CIN, COUT, T, KW = 16, 64, 64, 4096, 64
FQ = 1024
TOUT = 2 * (FQ - 1)
TPAD = 2048

OB = 8
FB = 512


@functools.lru_cache(maxsize=1)
def _bases():
    t = np.arange(T, dtype=np.float64)[:, None]
    f = np.arange(FQ, dtype=np.float64)[None, :]
    ang = 2.0 * np.pi * t * f / T
    fwd = np.concatenate([np.cos(ang), -np.sin(ang)], axis=1)

    k = np.arange(KW, dtype=np.float64)[:, None]
    angw = 2.0 * np.pi * (T - KW + k) * f / T
    fwdw = np.concatenate([np.cos(angw), -np.sin(angw)], axis=1)

    tt = np.arange(TOUT, dtype=np.float64)[None, :]
    fi = np.arange(FQ, dtype=np.float64)[:, None]
    angi = 2.0 * np.pi * fi * tt / TOUT
    alpha = np.full((FQ, 1), 2.0)
    alpha[0, 0] = 1.0
    alpha[FQ - 1, 0] = 1.0
    ar = alpha * np.cos(angi) / TOUT
    ai = -2.0 * np.sin(angi) / TOUT
    ai[0, :] = 0.0
    ai[FQ - 1, :] = 0.0
    inv = np.concatenate([ar, ai], axis=0)
    inv = np.pad(inv, ((0, 0), (0, TPAD - TOUT)))
    return (fwd.astype(np.float32), fwdw.astype(np.float32),
            inv.astype(np.float32))


def _matmul_kernel(x_ref, w_ref, o_ref):
    o_ref[...] = jnp.dot(x_ref[...], w_ref[...],
                         preferred_element_type=jnp.float32)


def _mm(x, w, bm, bn, out_dtype=jnp.float32):
    m, k = x.shape
    _, n = w.shape
    return pl.pallas_call(
        _matmul_kernel,
        grid=(m // bm, n // bn),
        in_specs=[
            pl.BlockSpec((bm, k), lambda i, j: (i, 0)),
            pl.BlockSpec((k, bn), lambda i, j: (0, j)),
        ],
        out_specs=pl.BlockSpec((bm, bn), lambda i, j: (i, j)),
        out_shape=jax.ShapeDtypeStruct((m, n), out_dtype),
        compiler_params=pltpu.CompilerParams(
            dimension_semantics=("parallel", "arbitrary"),
        ),
    )(x, w)


def _hadamard_kernel(xr_ref, xi_ref, wr_ref, wi_ref, cr_ref, ci_ref):
    for oj in range(OB):
        acc_r = jnp.zeros((B, FB), jnp.float32)
        acc_i = jnp.zeros((B, FB), jnp.float32)
        for c in range(CIN):
            xr = xr_ref[c]
            xi = xi_ref[c]
            wr = wr_ref[oj * CIN + c]
            wi = wi_ref[oj * CIN + c]
            acc_r = acc_r + xr * wr - xi * wi
            acc_i = acc_i + xr * wi + xi * wr
        cr_ref[oj] = acc_r.astype(cr_ref.dtype)
        ci_ref[oj] = acc_i.astype(ci_ref.dtype)


def _hadamard(xhat, what):
    xv = xhat.reshape(CIN, B, 2 * FQ)
    wv = what.reshape(COUT * CIN, 1, 2 * FQ)
    nf = FQ // FB
    grid = (COUT // OB, nf)
    xspec = lambda off: pl.BlockSpec(
        (CIN, B, FB), lambda oi, fi: (0, 0, fi + off))
    wspec = lambda off: pl.BlockSpec(
        (OB * CIN, 1, FB), lambda oi, fi: (oi, 0, fi + off))
    ospec = pl.BlockSpec((OB, B, FB), lambda oi, fi: (oi, 0, fi))
    oshape = jax.ShapeDtypeStruct((COUT, B, FQ), jnp.bfloat16)
    return pl.pallas_call(
        _hadamard_kernel,
        grid=grid,
        in_specs=[xspec(0), xspec(nf), wspec(0), wspec(nf)],
        out_specs=[ospec, ospec],
        out_shape=[oshape, oshape],
        compiler_params=pltpu.CompilerParams(
            dimension_semantics=("parallel", "arbitrary"),
        ),
    )(xv, xv, wv, wv)


def _inverse_kernel(cr_ref, ci_ref, ar_ref, ai_ref, b_ref, o_ref):
    acc = jnp.dot(cr_ref[...], ar_ref[...],
                  preferred_element_type=jnp.float32)
    acc += jnp.dot(ci_ref[...], ai_ref[...],
                   preferred_element_type=jnp.float32)
    o_ref[...] = acc + pltpu.repeat(b_ref[...], o_ref.shape[1] // 128, axis=1)


def _inverse(cr, ci, inv_r, inv_i, bias_plane):
    m = COUT * B
    bm, bn = 512, 1024
    return pl.pallas_call(
        _inverse_kernel,
        grid=(m // bm, TPAD // bn),
        in_specs=[
            pl.BlockSpec((bm, FQ), lambda i, j: (i, 0)),
            pl.BlockSpec((bm, FQ), lambda i, j: (i, 0)),
            pl.BlockSpec((FQ, bn), lambda i, j: (0, j)),
            pl.BlockSpec((FQ, bn), lambda i, j: (0, j)),
            pl.BlockSpec((bm, 128), lambda i, j: (i, 0)),
        ],
        out_specs=pl.BlockSpec((bm, bn), lambda i, j: (i, j)),
        out_shape=jax.ShapeDtypeStruct((m, TPAD), jnp.float32),
        compiler_params=pltpu.CompilerParams(
            dimension_semantics=("parallel", "arbitrary"),
        ),
    )(cr, ci, inv_r, inv_i, bias_plane)


def kernel(x, weight, bias):
    fwd, fwdw, inv = _bases()
    fwd = jnp.asarray(fwd, jnp.bfloat16)
    fwdw = jnp.asarray(fwdw, jnp.bfloat16)
    inv_r = jnp.asarray(inv[:FQ], jnp.bfloat16)
    inv_i = jnp.asarray(inv[FQ:], jnp.bfloat16)

    xt = x.transpose(1, 0, 2).reshape(CIN * B, T).astype(jnp.bfloat16)
    xhat = _mm(xt, fwd, bm=512, bn=1024)

    w2 = weight.reshape(COUT * CIN, KW).astype(jnp.bfloat16)
    what = _mm(w2, fwdw, bm=1024, bn=2048)

    cr, ci = _hadamard(xhat, what)

    bias_plane = jnp.broadcast_to(
        jnp.repeat(bias, B)[:, None], (COUT * B, 128)).astype(jnp.float32)
    out = _inverse(cr.reshape(COUT * B, FQ), ci.reshape(COUT * B, FQ),
                   inv_r, inv_i, bias_plane)

    return out[:, :TOUT].reshape(COUT, B, TOUT).transpose(1, 0, 2)
